# baseline (speedup 1.0000x reference)
import functools
import math

import jax
import jax.numpy as jnp
from jax import lax
from jax.experimental import pallas as pl

SMOOTH = 0.1
PAD = 0

BR = 64
BC = 32000


def _loss_kernel(tgt_ref, x_ref, o_ref, *, eps, const):
    i = pl.program_id(0)
    j = pl.program_id(1)

    @pl.when(jnp.logical_and(i == 0, j == 0))
    def _():
        o_ref[...] = jnp.zeros((1, 1), jnp.float32)

    x = x_ref[...]
    t = tgt_ref[...].astype(jnp.int32)
    cols = lax.broadcasted_iota(jnp.int32, (BR, BC), 1) + j * BC
    pts = jnp.sum(jnp.where(cols == t, x, 0.0), axis=1)
    rs = jnp.sum(x, axis=1)
    mask = t[:, 0] != PAD
    part = jnp.sum(jnp.where(mask, -eps * rs + (eps - 0.9) * pts, 0.0))
    first = (j == 0).astype(jnp.float32)
    part = part + first * jnp.sum(jnp.where(mask, const + eps * x[:, 0], 0.0))
    o_ref[...] += part.reshape(1, 1)


def kernel(predicted_tensor, target_tensor):
    B, S, V = predicted_tensor.shape
    N = B * S
    pred = predicted_tensor.reshape(N, V)
    tgt = target_tensor.reshape(N, 1).astype(jnp.int32)

    eps = SMOOTH / (V - 2)
    const = SMOOTH * math.log(eps) + (1.0 - SMOOTH) * math.log(1.0 - SMOOTH)

    out = pl.pallas_call(
        functools.partial(_loss_kernel, eps=eps, const=const),
        grid=(N // BR, V // BC),
        in_specs=[
            pl.BlockSpec((BR, 1), lambda i, j: (i, 0)),
            pl.BlockSpec((BR, BC), lambda i, j: (i, j)),
        ],
        out_specs=pl.BlockSpec((1, 1), lambda i, j: (0, 0)),
        out_shape=jax.ShapeDtypeStruct((1, 1), jnp.float32),
    )(tgt, pred)
    return out[0, 0]


# speedup vs baseline: 1.0241x; 1.0241x over previous
import functools
import math

import jax
import jax.numpy as jnp
from jax import lax
from jax.experimental import pallas as pl
from jax.experimental.pallas import tpu as pltpu

SMOOTH = 0.1
PAD = 0

BR = 256
BC = 32000


def _loss_kernel(tgt_ref, x_ref, o_ref, *, eps, const):
    i = pl.program_id(0)
    j = pl.program_id(1)

    @pl.when(jnp.logical_and(i == 0, j == 0))
    def _():
        o_ref[...] = jnp.zeros((1, 1), jnp.float32)

    x = x_ref[...]
    t = tgt_ref[...].astype(jnp.int32)
    cols = lax.broadcasted_iota(jnp.int32, (BR, BC), 1) + j * BC
    pts = jnp.sum(jnp.where(cols == t, x, 0.0), axis=1)
    rs = jnp.sum(x, axis=1)
    mask = t[:, 0] != PAD
    part = jnp.sum(jnp.where(mask, -eps * rs + (eps - 0.9) * pts, 0.0))
    first = (j == 0).astype(jnp.float32)
    part = part + first * jnp.sum(jnp.where(mask, const + eps * x[:, 0], 0.0))
    o_ref[...] += part.reshape(1, 1)


def kernel(predicted_tensor, target_tensor):
    B, S, V = predicted_tensor.shape
    N = B * S
    pred = predicted_tensor.reshape(N, V)
    tgt = target_tensor.reshape(N, 1).astype(jnp.int32)

    eps = SMOOTH / (V - 2)
    const = SMOOTH * math.log(eps) + (1.0 - SMOOTH) * math.log(1.0 - SMOOTH)

    out = pl.pallas_call(
        functools.partial(_loss_kernel, eps=eps, const=const),
        grid=(N // BR, V // BC),
        in_specs=[
            pl.BlockSpec((BR, 1), lambda i, j: (i, 0)),
            pl.BlockSpec((BR, BC), lambda i, j: (i, j)),
        ],
        out_specs=pl.BlockSpec((1, 1), lambda i, j: (0, 0)),
        out_shape=jax.ShapeDtypeStruct((1, 1), jnp.float32),
        compiler_params=pltpu.CompilerParams(vmem_limit_bytes=100 * 1024 * 1024),
    )(tgt, pred)
    return out[0, 0]


# speedup vs baseline: 1.0615x; 1.0366x over previous
import functools
import math

import jax
import jax.numpy as jnp
from jax import lax
from jax.experimental import pallas as pl

SMOOTH = 0.1
PAD = 0

BR = 128
BC = 32000


def _loss_kernel(tgt_ref, x_ref, o_ref, *, eps, const):
    i = pl.program_id(0)
    j = pl.program_id(1)

    @pl.when(jnp.logical_and(i == 0, j == 0))
    def _():
        o_ref[...] = jnp.zeros((1, 1), jnp.float32)

    x = x_ref[...]
    t = tgt_ref[...].astype(jnp.int32)
    cols = lax.broadcasted_iota(jnp.int32, (BR, BC), 1) + j * BC
    pts = jnp.sum(jnp.where(cols == t, x, 0.0), axis=1)
    rs = jnp.sum(x, axis=1)
    mask = t[:, 0] != PAD
    part = jnp.sum(jnp.where(mask, -eps * rs + (eps - 0.9) * pts, 0.0))
    first = (j == 0).astype(jnp.float32)
    part = part + first * jnp.sum(jnp.where(mask, const + eps * x[:, 0], 0.0))
    o_ref[...] += part.reshape(1, 1)


def kernel(predicted_tensor, target_tensor):
    B, S, V = predicted_tensor.shape
    N = B * S
    pred = predicted_tensor.reshape(N, V)
    tgt = target_tensor.reshape(N, 1).astype(jnp.int32)

    eps = SMOOTH / (V - 2)
    const = SMOOTH * math.log(eps) + (1.0 - SMOOTH) * math.log(1.0 - SMOOTH)

    out = pl.pallas_call(
        functools.partial(_loss_kernel, eps=eps, const=const),
        grid=(N // BR, V // BC),
        in_specs=[
            pl.BlockSpec((BR, 1), lambda i, j: (i, 0)),
            pl.BlockSpec((BR, BC), lambda i, j: (i, j)),
        ],
        out_specs=pl.BlockSpec((1, 1), lambda i, j: (0, 0)),
        out_shape=jax.ShapeDtypeStruct((1, 1), jnp.float32),
    )(tgt, pred)
    return out[0, 0]


# speedup vs baseline: 1.0628x; 1.0012x over previous
import functools
import math

import jax
import jax.numpy as jnp
from jax import lax
from jax.experimental import pallas as pl

SMOOTH = 0.1
PAD = 0

BR = 128
BC = 32000


HALF = BC // 2


def _loss_kernel(tgt_ref, xa_ref, xb_ref, o_ref, *, eps, const):
    i = pl.program_id(0)

    @pl.when(i == 0)
    def _():
        o_ref[...] = jnp.zeros((1, 1), jnp.float32)

    xa = xa_ref[...]
    xb = xb_ref[...]
    t = tgt_ref[...].astype(jnp.int32)
    cols = lax.broadcasted_iota(jnp.int32, (BR, HALF), 1)
    pts = jnp.sum(jnp.where(cols == t, xa, 0.0), axis=1)
    pts = pts + jnp.sum(jnp.where(cols + HALF == t, xb, 0.0), axis=1)
    rs = jnp.sum(xa, axis=1) + jnp.sum(xb, axis=1)
    mask = t[:, 0] != PAD
    part = jnp.sum(jnp.where(mask, -eps * rs + (eps - 0.9) * pts, 0.0))
    part = part + jnp.sum(jnp.where(mask, const + eps * xa[:, 0], 0.0))
    o_ref[...] += part.reshape(1, 1)


def kernel(predicted_tensor, target_tensor):
    B, S, V = predicted_tensor.shape
    N = B * S
    pred = predicted_tensor.reshape(N, V)
    tgt = target_tensor.reshape(N, 1).astype(jnp.int32)

    eps = SMOOTH / (V - 2)
    const = SMOOTH * math.log(eps) + (1.0 - SMOOTH) * math.log(1.0 - SMOOTH)

    out = pl.pallas_call(
        functools.partial(_loss_kernel, eps=eps, const=const),
        grid=(N // BR,),
        in_specs=[
            pl.BlockSpec((BR, 1), lambda i: (i, 0)),
            pl.BlockSpec((BR, HALF), lambda i: (i, 0)),
            pl.BlockSpec((BR, HALF), lambda i: (i, 1)),
        ],
        out_specs=pl.BlockSpec((1, 1), lambda i: (0, 0)),
        out_shape=jax.ShapeDtypeStruct((1, 1), jnp.float32),
    )(tgt, pred, pred)
    return out[0, 0]
